# baseline (speedup 1.0000x reference)
import functools

import jax
import jax.numpy as jnp
from jax import lax
from jax.experimental import pallas as pl
from jax.experimental.pallas import tpu as pltpu
from jax.experimental.pallas import tpu_sc as plsc

B, S, H = 1, 2048, 1024
I = 768
E = 8
K = 2
T = B * S
BM = 256
NBLK = T * K // BM + E
NPAD = NBLK * BM
NSC = 32
TCH = T // NSC


def _route_body(x_ref, wr_ref, logits_ref, xbf_ref, s1_ref, s2_ref, w1_ref,
                w2_ref, ebid_ref, counts_ref):
    x = x_ref[...]
    u = lax.bitcast_convert_type(x, jnp.int32)
    r = (u + 0x7FFF + ((u >> 16) & 1)) >> 16
    packed = (r[:, :H // 2] & 0xFFFF) | (r[:, H // 2:] << 16)
    xbf_ref[...] = lax.bitcast_convert_type(packed, jnp.float32)
    logits = jnp.dot(x, wr_ref[...], preferred_element_type=jnp.float32)
    logits_ref[...] = logits
    m = jnp.max(logits, axis=-1, keepdims=True)
    ex = jnp.exp(logits - m)
    rw = ex / jnp.sum(ex, axis=-1, keepdims=True)
    idx = lax.broadcasted_iota(jnp.int32, (T, E), 1)
    m1 = jnp.max(rw, axis=-1, keepdims=True)
    a1 = jnp.min(jnp.where(rw == m1, idx, E), axis=-1, keepdims=True)
    mask1 = idx == a1
    rw2 = jnp.where(mask1, -1.0, rw)
    m2 = jnp.max(rw2, axis=-1, keepdims=True)
    a2 = jnp.min(jnp.where(rw2 == m2, idx, E), axis=-1, keepdims=True)
    mask2 = idx == a2
    denom = m1 + m2
    w1_ref[...] = jnp.broadcast_to(m1 / denom, (T, 16))
    w2_ref[...] = jnp.broadcast_to(m2 / denom, (T, 16))

    msum = (mask1 | mask2).astype(jnp.float32)
    ri = lax.broadcasted_iota(jnp.int32, (T, T), 0)
    cj = lax.broadcasted_iota(jnp.int32, (T, T), 1)
    ltri = (ri >= cj).astype(jnp.bfloat16)
    csum = jnp.dot(ltri, msum.astype(jnp.bfloat16),
                   preferred_element_type=jnp.float32)
    counts = csum[T - 1:T, :]
    counts_ref[...] = counts
    rank = csum - msum

    ci = counts.astype(jnp.int32)
    padded = ((ci + (BM - 1)) // BM) * BM
    offs = []
    acc = jnp.zeros((1, 1), jnp.int32)
    for e in range(E):
        offs.append(acc)
        acc = acc + padded[:, e:e + 1]
    pad_off = jnp.concatenate(offs, axis=1)

    slot_of = pad_off.astype(jnp.float32) + rank
    s1_ref[...] = jnp.sum(jnp.where(mask1, slot_of, 0.0), axis=-1,
                          keepdims=True).astype(jnp.int32)
    s2_ref[...] = jnp.sum(jnp.where(mask2, slot_of, 0.0), axis=-1,
                          keepdims=True).astype(jnp.int32)

    blk = lax.broadcasted_iota(jnp.int32, (1, NBLK), 1) * BM
    eb = jnp.zeros((1, NBLK), jnp.int32)
    for e in range(E):
        eb = eb + (blk >= pad_off[:, e:e + 1]).astype(jnp.int32)
    ebid_ref[...] = eb - 1


def _route(xf, W_router):
    return pl.pallas_call(
        _route_body,
        out_shape=[
            jax.ShapeDtypeStruct((T, E), jnp.float32),
            jax.ShapeDtypeStruct((T, H // 2), jnp.float32),
            jax.ShapeDtypeStruct((T, 1), jnp.int32),
            jax.ShapeDtypeStruct((T, 1), jnp.int32),
            jax.ShapeDtypeStruct((T, 16), jnp.float32),
            jax.ShapeDtypeStruct((T, 16), jnp.float32),
            jax.ShapeDtypeStruct((1, NBLK), jnp.int32),
            jax.ShapeDtypeStruct((1, E), jnp.float32),
        ],
    )(xf, W_router)


_sc_mesh = plsc.VectorSubcoreMesh(core_axis_name="c", subcore_axis_name="s")


@functools.partial(
    pl.kernel,
    mesh=_sc_mesh,
    out_type=jax.ShapeDtypeStruct((NPAD, H // 2), jnp.float32),
    scratch_types=[
        pltpu.VMEM((TCH, H // 2), jnp.float32),
        pltpu.VMEM((TCH,), jnp.int32),
        pltpu.VMEM((TCH,), jnp.int32),
        pltpu.SemaphoreType.DMA,
        pltpu.SemaphoreType.DMA,
    ],
)
def _dispatch(x_hbm, s1_hbm, s2_hbm, xs_hbm, xrows, i1, i2, sem1, sem2):
    wid = lax.axis_index("s") * 2 + lax.axis_index("c")
    base = wid * TCH
    pltpu.sync_copy(x_hbm.at[pl.ds(base, TCH)], xrows)
    pltpu.sync_copy(s1_hbm.at[pl.ds(base, TCH)], i1)
    pltpu.sync_copy(s2_hbm.at[pl.ds(base, TCH)], i2)
    c1 = pltpu.async_copy(xrows, xs_hbm.at[i1], sem1)
    c2 = pltpu.async_copy(xrows, xs_hbm.at[i2], sem2)
    c1.wait()
    c2.wait()


def _experts_body(eb_ref, xs_ref, wup_ref, bup_ref, wdn_ref, bdn_ref,
                  ys_ref):
    p = lax.bitcast_convert_type(xs_ref[...], jnp.int32)
    xlo = lax.bitcast_convert_type(p << 16, jnp.float32)
    xhi = lax.bitcast_convert_type(p & jnp.int32(-65536), jnp.float32)
    xb = jnp.concatenate([xlo, xhi], axis=1).astype(jnp.bfloat16)
    up = jnp.dot(xb, wup_ref[0], preferred_element_type=jnp.float32)
    up = up + bup_ref[0]
    act = up * jax.nn.sigmoid(up)
    dn = jnp.dot(act.astype(jnp.bfloat16), wdn_ref[0],
                 preferred_element_type=jnp.float32)
    ys_ref[...] = dn + bdn_ref[0]


def _experts(ebid, xs, W_up, b_up, W_down, b_down):
    grid_spec = pltpu.PrefetchScalarGridSpec(
        num_scalar_prefetch=1,
        grid=(NBLK,),
        in_specs=[
            pl.BlockSpec((BM, H // 2), lambda b, eb: (b, 0)),
            pl.BlockSpec((1, H, I), lambda b, eb: (eb[b], 0, 0)),
            pl.BlockSpec((1, 1, I), lambda b, eb: (eb[b], 0, 0)),
            pl.BlockSpec((1, I, H), lambda b, eb: (eb[b], 0, 0)),
            pl.BlockSpec((1, 1, H), lambda b, eb: (eb[b], 0, 0)),
        ],
        out_specs=pl.BlockSpec((BM, H), lambda b, eb: (b, 0)),
    )
    return pl.pallas_call(
        _experts_body,
        grid_spec=grid_spec,
        out_shape=jax.ShapeDtypeStruct((NPAD, H), jnp.float32),
    )(ebid, xs, W_up.astype(jnp.bfloat16), b_up.reshape(E, 1, I),
      W_down.astype(jnp.bfloat16), b_down.reshape(E, 1, H))


_HALF = 32


@functools.partial(
    pl.kernel,
    mesh=_sc_mesh,
    out_type=jax.ShapeDtypeStruct((T, H), jnp.float32),
    scratch_types=[
        pltpu.VMEM((_HALF, H), jnp.float32),
        pltpu.VMEM((_HALF, H), jnp.float32),
        pltpu.VMEM((TCH,), jnp.int32),
        pltpu.VMEM((TCH,), jnp.int32),
        pltpu.VMEM((TCH, 16), jnp.float32),
        pltpu.VMEM((TCH, 16), jnp.float32),
        pltpu.SemaphoreType.DMA,
        pltpu.SemaphoreType.DMA,
    ],
)
def _combine(ys_hbm, s1_hbm, s2_hbm, w1_hbm, w2_hbm, out_hbm,
             y1v, y2v, i1, i2, v1, v2, sem1, sem2):
    wid = lax.axis_index("s") * 2 + lax.axis_index("c")
    base = wid * TCH
    pltpu.sync_copy(s1_hbm.at[pl.ds(base, TCH)], i1)
    pltpu.sync_copy(s2_hbm.at[pl.ds(base, TCH)], i2)
    pltpu.sync_copy(w1_hbm.at[pl.ds(base, TCH)], v1)
    pltpu.sync_copy(w2_hbm.at[pl.ds(base, TCH)], v2)
    for half in range(TCH // _HALF):
        hb = base + half * _HALF
        c1 = pltpu.async_copy(ys_hbm.at[i1.at[pl.ds(half * _HALF, _HALF)]],
                              y1v, sem1)
        c2 = pltpu.async_copy(ys_hbm.at[i2.at[pl.ds(half * _HALF, _HALF)]],
                              y2v, sem2)
        c1.wait()
        c2.wait()

        def _add_row(r, _):
            t = half * _HALF + r
            wv1 = v1[t]
            wv2 = v2[t]
            for c in range(0, H, 16):
                y1v[r, pl.ds(c, 16)] = (y1v[r, pl.ds(c, 16)] * wv1
                                        + y2v[r, pl.ds(c, 16)] * wv2)
            return 0

        lax.fori_loop(0, _HALF, _add_row, 0)
        pltpu.sync_copy(y1v, out_hbm.at[pl.ds(hb, _HALF)])


@jax.jit
def _moe_sparse(xf, W_router, W_up, b_up, W_down, b_down):
    logits, xbf, s1, s2, w1, w2, ebid, counts = _route(xf, W_router)
    s1f = s1.reshape(T)
    s2f = s2.reshape(T)
    xs = _dispatch(xbf, s1f, s2f)
    ys = _experts(ebid.reshape(NBLK), xs, W_up, b_up, W_down, b_down)
    out = _combine(ys, s1f, s2f, w1, w2)
    return out, logits, counts


def kernel(hidden_states, W_router, W_up, b_up, W_down, b_down):
    xf = hidden_states.reshape(T, H)
    out, logits, counts = _moe_sparse(xf, W_router, W_up, b_up, W_down,
                                      b_down)
    usage = counts[0] * (E / (T * K))
    s = jnp.sum(usage)
    aux_loss = s * s / (E * E)
    return out.reshape(B, S, H), logits, aux_loss


# speedup vs baseline: 1.1286x; 1.1286x over previous
import functools

import jax
import jax.numpy as jnp
from jax import lax
from jax.experimental import pallas as pl
from jax.experimental.pallas import tpu as pltpu
from jax.experimental.pallas import tpu_sc as plsc

B, S, H = 1, 2048, 1024
I = 768
E = 8
K = 2
T = B * S
BM = 256
NBLK = T * K // BM + E
NPAD = NBLK * BM
NSC = 32
TCH = T // NSC


def _route_body(x_ref, wr_ref, logits_ref, xbf_ref, s1_ref, s2_ref, w1_ref,
                w2_ref, ebid_ref, counts_ref):
    x = x_ref[...]
    u = lax.bitcast_convert_type(x, jnp.int32)
    r = (u + 0x7FFF + ((u >> 16) & 1)) >> 16
    packed = (r[:, :H // 2] & 0xFFFF) | (r[:, H // 2:] << 16)
    xbf_ref[...] = lax.bitcast_convert_type(packed, jnp.float32)
    logits = jnp.dot(x, wr_ref[...], preferred_element_type=jnp.float32)
    logits_ref[...] = logits
    m = jnp.max(logits, axis=-1, keepdims=True)
    ex = jnp.exp(logits - m)
    rw = ex / jnp.sum(ex, axis=-1, keepdims=True)
    idx = lax.broadcasted_iota(jnp.int32, (T, E), 1)
    m1 = jnp.max(rw, axis=-1, keepdims=True)
    a1 = jnp.min(jnp.where(rw == m1, idx, E), axis=-1, keepdims=True)
    mask1 = idx == a1
    rw2 = jnp.where(mask1, -1.0, rw)
    m2 = jnp.max(rw2, axis=-1, keepdims=True)
    a2 = jnp.min(jnp.where(rw2 == m2, idx, E), axis=-1, keepdims=True)
    mask2 = idx == a2
    denom = m1 + m2
    w1_ref[...] = jnp.broadcast_to(m1 / denom, (T, 16))
    w2_ref[...] = jnp.broadcast_to(m2 / denom, (T, 16))

    msum = (mask1 | mask2).astype(jnp.float32)
    ri = lax.broadcasted_iota(jnp.int32, (T, T), 0)
    cj = lax.broadcasted_iota(jnp.int32, (T, T), 1)
    ltri = (ri >= cj).astype(jnp.bfloat16)
    csum = jnp.dot(ltri, msum.astype(jnp.bfloat16),
                   preferred_element_type=jnp.float32)
    counts = csum[T - 1:T, :]
    counts_ref[...] = counts
    rank = csum - msum

    ci = counts.astype(jnp.int32)
    padded = ((ci + (BM - 1)) // BM) * BM
    offs = []
    acc = jnp.zeros((1, 1), jnp.int32)
    for e in range(E):
        offs.append(acc)
        acc = acc + padded[:, e:e + 1]
    pad_off = jnp.concatenate(offs, axis=1)

    slot_of = pad_off.astype(jnp.float32) + rank
    s1_ref[...] = jnp.sum(jnp.where(mask1, slot_of, 0.0), axis=-1,
                          keepdims=True).astype(jnp.int32)
    s2_ref[...] = jnp.sum(jnp.where(mask2, slot_of, 0.0), axis=-1,
                          keepdims=True).astype(jnp.int32)

    blk = lax.broadcasted_iota(jnp.int32, (1, NBLK), 1) * BM
    eb = jnp.zeros((1, NBLK), jnp.int32)
    for e in range(E):
        eb = eb + (blk >= pad_off[:, e:e + 1]).astype(jnp.int32)
    ebid_ref[...] = eb - 1


def _route(xf, W_router):
    return pl.pallas_call(
        _route_body,
        out_shape=[
            jax.ShapeDtypeStruct((T, E), jnp.float32),
            jax.ShapeDtypeStruct((T, H // 2), jnp.float32),
            jax.ShapeDtypeStruct((T, 1), jnp.int32),
            jax.ShapeDtypeStruct((T, 1), jnp.int32),
            jax.ShapeDtypeStruct((T, 16), jnp.float32),
            jax.ShapeDtypeStruct((T, 16), jnp.float32),
            jax.ShapeDtypeStruct((1, NBLK), jnp.int32),
            jax.ShapeDtypeStruct((1, E), jnp.float32),
        ],
    )(xf, W_router)


_sc_mesh = plsc.VectorSubcoreMesh(core_axis_name="c", subcore_axis_name="s")


@functools.partial(
    pl.kernel,
    mesh=_sc_mesh,
    out_type=jax.ShapeDtypeStruct((NPAD, H // 2), jnp.float32),
    scratch_types=[
        pltpu.VMEM((TCH, H // 2), jnp.float32),
        pltpu.VMEM((TCH,), jnp.int32),
        pltpu.VMEM((TCH,), jnp.int32),
        pltpu.SemaphoreType.DMA,
        pltpu.SemaphoreType.DMA,
    ],
)
def _dispatch(x_hbm, s1_hbm, s2_hbm, xs_hbm, xrows, i1, i2, sem1, sem2):
    wid = lax.axis_index("s") * 2 + lax.axis_index("c")
    base = wid * TCH
    pltpu.sync_copy(x_hbm.at[pl.ds(base, TCH)], xrows)
    pltpu.sync_copy(s1_hbm.at[pl.ds(base, TCH)], i1)
    pltpu.sync_copy(s2_hbm.at[pl.ds(base, TCH)], i2)
    c1 = pltpu.async_copy(xrows, xs_hbm.at[i1], sem1)
    c2 = pltpu.async_copy(xrows, xs_hbm.at[i2], sem2)
    c1.wait()
    c2.wait()


def _experts_body(eb_ref, xs_ref, wup_ref, bup_ref, wdn_ref, bdn_ref,
                  ys_ref):
    p = lax.bitcast_convert_type(xs_ref[...], jnp.int32)
    xlo = lax.bitcast_convert_type(p << 16, jnp.float32)
    xhi = lax.bitcast_convert_type(p & jnp.int32(-65536), jnp.float32)
    xb = jnp.concatenate([xlo, xhi], axis=1).astype(jnp.bfloat16)
    up = jnp.dot(xb, wup_ref[0].astype(jnp.bfloat16),
                 preferred_element_type=jnp.float32)
    up = up + bup_ref[0]
    act = up * jax.nn.sigmoid(up)
    dn = jnp.dot(act.astype(jnp.bfloat16), wdn_ref[0].astype(jnp.bfloat16),
                 preferred_element_type=jnp.float32)
    ys_ref[...] = dn + bdn_ref[0]


def _experts(ebid, xs, W_up, b_up, W_down, b_down):
    grid_spec = pltpu.PrefetchScalarGridSpec(
        num_scalar_prefetch=1,
        grid=(NBLK,),
        in_specs=[
            pl.BlockSpec((BM, H // 2), lambda b, eb: (b, 0)),
            pl.BlockSpec((1, H, I), lambda b, eb: (eb[b], 0, 0)),
            pl.BlockSpec((1, 1, I), lambda b, eb: (eb[b], 0, 0)),
            pl.BlockSpec((1, I, H), lambda b, eb: (eb[b], 0, 0)),
            pl.BlockSpec((1, 1, H), lambda b, eb: (eb[b], 0, 0)),
        ],
        out_specs=pl.BlockSpec((BM, H), lambda b, eb: (b, 0)),
    )
    return pl.pallas_call(
        _experts_body,
        grid_spec=grid_spec,
        out_shape=jax.ShapeDtypeStruct((NPAD, H), jnp.float32),
    )(ebid, xs, W_up, b_up.reshape(E, 1, I), W_down, b_down.reshape(E, 1, H))


_HALF = 32


@functools.partial(
    pl.kernel,
    mesh=_sc_mesh,
    out_type=jax.ShapeDtypeStruct((T, H), jnp.float32),
    scratch_types=[
        pltpu.VMEM((_HALF, H), jnp.float32),
        pltpu.VMEM((_HALF, H), jnp.float32),
        pltpu.VMEM((TCH,), jnp.int32),
        pltpu.VMEM((TCH,), jnp.int32),
        pltpu.VMEM((TCH, 16), jnp.float32),
        pltpu.VMEM((TCH, 16), jnp.float32),
        pltpu.SemaphoreType.DMA,
        pltpu.SemaphoreType.DMA,
    ],
)
def _combine(ys_hbm, s1_hbm, s2_hbm, w1_hbm, w2_hbm, out_hbm,
             y1v, y2v, i1, i2, v1, v2, sem1, sem2):
    wid = lax.axis_index("s") * 2 + lax.axis_index("c")
    base = wid * TCH
    pltpu.sync_copy(s1_hbm.at[pl.ds(base, TCH)], i1)
    pltpu.sync_copy(s2_hbm.at[pl.ds(base, TCH)], i2)
    pltpu.sync_copy(w1_hbm.at[pl.ds(base, TCH)], v1)
    pltpu.sync_copy(w2_hbm.at[pl.ds(base, TCH)], v2)
    for half in range(TCH // _HALF):
        hb = base + half * _HALF
        c1 = pltpu.async_copy(ys_hbm.at[i1.at[pl.ds(half * _HALF, _HALF)]],
                              y1v, sem1)
        c2 = pltpu.async_copy(ys_hbm.at[i2.at[pl.ds(half * _HALF, _HALF)]],
                              y2v, sem2)
        c1.wait()
        c2.wait()

        def _add_row(r, _):
            t = half * _HALF + r
            wv1 = v1[t]
            wv2 = v2[t]
            for c in range(0, H, 16):
                y1v[r, pl.ds(c, 16)] = (y1v[r, pl.ds(c, 16)] * wv1
                                        + y2v[r, pl.ds(c, 16)] * wv2)
            return 0

        lax.fori_loop(0, _HALF, _add_row, 0)
        pltpu.sync_copy(y1v, out_hbm.at[pl.ds(hb, _HALF)])


@jax.jit
def _moe_sparse(xf, W_router, W_up, b_up, W_down, b_down):
    logits, xbf, s1, s2, w1, w2, ebid, counts = _route(xf, W_router)
    s1f = s1.reshape(T)
    s2f = s2.reshape(T)
    xs = _dispatch(xbf, s1f, s2f)
    ys = _experts(ebid.reshape(NBLK), xs, W_up, b_up, W_down, b_down)
    out = _combine(ys, s1f, s2f, w1, w2)
    return out, logits, counts


def kernel(hidden_states, W_router, W_up, b_up, W_down, b_down):
    xf = hidden_states.reshape(T, H)
    out, logits, counts = _moe_sparse(xf, W_router, W_up, b_up, W_down,
                                      b_down)
    usage = counts[0] * (E / (T * K))
    s = jnp.sum(usage)
    aux_loss = s * s / (E * E)
    return out.reshape(B, S, H), logits, aux_loss


# speedup vs baseline: 1.1554x; 1.0237x over previous
import functools

import jax
import jax.numpy as jnp
from jax import lax
from jax.experimental import pallas as pl
from jax.experimental.pallas import tpu as pltpu
from jax.experimental.pallas import tpu_sc as plsc

B, S, H = 1, 2048, 1024
I = 768
E = 8
K = 2
T = B * S
BM = 256
NBLK = T * K // BM + E
NPAD = NBLK * BM
NSC = 32
TCH = T // NSC


def _route_body(x_ref, wr_ref, logits_ref, xbf_ref, s1_ref, s2_ref, w1_ref,
                w2_ref, ebid_ref, counts_ref):
    x = x_ref[...]
    u = lax.bitcast_convert_type(x, jnp.int32)
    r = (u + 0x7FFF + ((u >> 16) & 1)) >> 16
    packed = (r[:, :H // 2] & 0xFFFF) | (r[:, H // 2:] << 16)
    xbf_ref[...] = lax.bitcast_convert_type(packed, jnp.float32)
    logits = jnp.dot(x, wr_ref[...], preferred_element_type=jnp.float32)
    logits_ref[...] = logits
    m = jnp.max(logits, axis=-1, keepdims=True)
    ex = jnp.exp(logits - m)
    rw = ex / jnp.sum(ex, axis=-1, keepdims=True)
    idx = lax.broadcasted_iota(jnp.int32, (T, E), 1)
    m1 = jnp.max(rw, axis=-1, keepdims=True)
    a1 = jnp.min(jnp.where(rw == m1, idx, E), axis=-1, keepdims=True)
    mask1 = idx == a1
    rw2 = jnp.where(mask1, -1.0, rw)
    m2 = jnp.max(rw2, axis=-1, keepdims=True)
    a2 = jnp.min(jnp.where(rw2 == m2, idx, E), axis=-1, keepdims=True)
    mask2 = idx == a2
    denom = m1 + m2
    w1_ref[...] = jnp.broadcast_to(m1 / denom, (T, 16))
    w2_ref[...] = jnp.broadcast_to(m2 / denom, (T, 16))

    msum = (mask1 | mask2).astype(jnp.float32)
    ri = lax.broadcasted_iota(jnp.int32, (T, T), 0)
    cj = lax.broadcasted_iota(jnp.int32, (T, T), 1)
    ltri = (ri >= cj).astype(jnp.bfloat16)
    csum = jnp.dot(ltri, msum.astype(jnp.bfloat16),
                   preferred_element_type=jnp.float32)
    counts = csum[T - 1:T, :]
    counts_ref[...] = counts
    rank = csum - msum

    ci = counts.astype(jnp.int32)
    padded = ((ci + (BM - 1)) // BM) * BM
    offs = []
    acc = jnp.zeros((1, 1), jnp.int32)
    for e in range(E):
        offs.append(acc)
        acc = acc + padded[:, e:e + 1]
    pad_off = jnp.concatenate(offs, axis=1)

    slot_of = pad_off.astype(jnp.float32) + rank
    s1_ref[...] = jnp.sum(jnp.where(mask1, slot_of, 0.0),
                          axis=-1).astype(jnp.int32)
    s2_ref[...] = jnp.sum(jnp.where(mask2, slot_of, 0.0),
                          axis=-1).astype(jnp.int32)

    blk = lax.broadcasted_iota(jnp.int32, (1, NBLK), 1) * BM
    eb = jnp.zeros((1, NBLK), jnp.int32)
    for e in range(E):
        eb = eb + (blk >= pad_off[:, e:e + 1]).astype(jnp.int32)
    ebid_ref[...] = (eb - 1).reshape(NBLK)


def _route(xf, W_router):
    return pl.pallas_call(
        _route_body,
        out_shape=[
            jax.ShapeDtypeStruct((T, E), jnp.float32),
            jax.ShapeDtypeStruct((T, H // 2), jnp.float32),
            jax.ShapeDtypeStruct((T,), jnp.int32),
            jax.ShapeDtypeStruct((T,), jnp.int32),
            jax.ShapeDtypeStruct((T, 16), jnp.float32),
            jax.ShapeDtypeStruct((T, 16), jnp.float32),
            jax.ShapeDtypeStruct((NBLK,), jnp.int32),
            jax.ShapeDtypeStruct((1, E), jnp.float32),
        ],
    )(xf, W_router)


_sc_mesh = plsc.VectorSubcoreMesh(core_axis_name="c", subcore_axis_name="s")


@functools.partial(
    pl.kernel,
    mesh=_sc_mesh,
    out_type=jax.ShapeDtypeStruct((NPAD, H // 2), jnp.float32),
    scratch_types=[
        pltpu.VMEM((TCH, H // 2), jnp.float32),
        pltpu.VMEM((TCH,), jnp.int32),
        pltpu.VMEM((TCH,), jnp.int32),
        pltpu.SemaphoreType.DMA,
        pltpu.SemaphoreType.DMA,
    ],
)
def _dispatch(x_hbm, s1_hbm, s2_hbm, xs_hbm, xrows, i1, i2, sem1, sem2):
    wid = lax.axis_index("s") * 2 + lax.axis_index("c")
    base = wid * TCH
    pltpu.sync_copy(x_hbm.at[pl.ds(base, TCH)], xrows)
    pltpu.sync_copy(s1_hbm.at[pl.ds(base, TCH)], i1)
    pltpu.sync_copy(s2_hbm.at[pl.ds(base, TCH)], i2)
    c1 = pltpu.async_copy(xrows, xs_hbm.at[i1], sem1)
    c2 = pltpu.async_copy(xrows, xs_hbm.at[i2], sem2)
    c1.wait()
    c2.wait()


def _experts_body(eb_ref, xs_ref, wup_ref, bup_ref, wdn_ref, bdn_ref,
                  ys_ref):
    p = lax.bitcast_convert_type(xs_ref[...], jnp.int32)
    xlo = lax.bitcast_convert_type(p << 16, jnp.float32)
    xhi = lax.bitcast_convert_type(p & jnp.int32(-65536), jnp.float32)
    xb = jnp.concatenate([xlo, xhi], axis=1).astype(jnp.bfloat16)
    up = jnp.dot(xb, wup_ref[0].astype(jnp.bfloat16),
                 preferred_element_type=jnp.float32)
    up = up + bup_ref[0]
    act = up * jax.nn.sigmoid(up)
    dn = jnp.dot(act.astype(jnp.bfloat16), wdn_ref[0].astype(jnp.bfloat16),
                 preferred_element_type=jnp.float32)
    ys_ref[...] = dn + bdn_ref[0]


def _experts(ebid, xs, W_up, b_up, W_down, b_down):
    grid_spec = pltpu.PrefetchScalarGridSpec(
        num_scalar_prefetch=1,
        grid=(NBLK,),
        in_specs=[
            pl.BlockSpec((BM, H // 2), lambda b, eb: (b, 0)),
            pl.BlockSpec((1, H, I), lambda b, eb: (eb[b], 0, 0)),
            pl.BlockSpec((1, 1, I), lambda b, eb: (eb[b], 0, 0)),
            pl.BlockSpec((1, I, H), lambda b, eb: (eb[b], 0, 0)),
            pl.BlockSpec((1, 1, H), lambda b, eb: (eb[b], 0, 0)),
        ],
        out_specs=pl.BlockSpec((BM, H), lambda b, eb: (b, 0)),
    )
    return pl.pallas_call(
        _experts_body,
        grid_spec=grid_spec,
        out_shape=jax.ShapeDtypeStruct((NPAD, H), jnp.float32),
    )(ebid, xs, W_up, b_up.reshape(E, 1, I), W_down, b_down.reshape(E, 1, H))


_HALF = 32


@functools.partial(
    pl.kernel,
    mesh=_sc_mesh,
    out_type=jax.ShapeDtypeStruct((T, H), jnp.float32),
    scratch_types=[
        pltpu.VMEM((_HALF, H), jnp.float32),
        pltpu.VMEM((_HALF, H), jnp.float32),
        pltpu.VMEM((TCH,), jnp.int32),
        pltpu.VMEM((TCH,), jnp.int32),
        pltpu.VMEM((TCH, 16), jnp.float32),
        pltpu.VMEM((TCH, 16), jnp.float32),
        pltpu.SemaphoreType.DMA,
        pltpu.SemaphoreType.DMA,
    ],
)
def _combine(ys_hbm, s1_hbm, s2_hbm, w1_hbm, w2_hbm, out_hbm,
             y1v, y2v, i1, i2, v1, v2, sem1, sem2):
    wid = lax.axis_index("s") * 2 + lax.axis_index("c")
    base = wid * TCH
    pltpu.sync_copy(s1_hbm.at[pl.ds(base, TCH)], i1)
    pltpu.sync_copy(s2_hbm.at[pl.ds(base, TCH)], i2)
    pltpu.sync_copy(w1_hbm.at[pl.ds(base, TCH)], v1)
    pltpu.sync_copy(w2_hbm.at[pl.ds(base, TCH)], v2)
    for half in range(TCH // _HALF):
        hb = base + half * _HALF
        c1 = pltpu.async_copy(ys_hbm.at[i1.at[pl.ds(half * _HALF, _HALF)]],
                              y1v, sem1)
        c2 = pltpu.async_copy(ys_hbm.at[i2.at[pl.ds(half * _HALF, _HALF)]],
                              y2v, sem2)
        c1.wait()
        c2.wait()

        def _add_row(r, _):
            t = half * _HALF + r
            wv1 = v1[t]
            wv2 = v2[t]
            for c in range(0, H, 16):
                y1v[r, pl.ds(c, 16)] = (y1v[r, pl.ds(c, 16)] * wv1
                                        + y2v[r, pl.ds(c, 16)] * wv2)
            return 0

        lax.fori_loop(0, _HALF, _add_row, 0)
        pltpu.sync_copy(y1v, out_hbm.at[pl.ds(hb, _HALF)])


@jax.jit
def _moe_sparse(xf, W_router, W_up, b_up, W_down, b_down):
    logits, xbf, s1, s2, w1, w2, ebid, counts = _route(xf, W_router)
    xs = _dispatch(xbf, s1, s2)
    ys = _experts(ebid, xs, W_up, b_up, W_down, b_down)
    out = _combine(ys, s1, s2, w1, w2)
    return out, logits, counts


def kernel(hidden_states, W_router, W_up, b_up, W_down, b_down):
    xf = hidden_states.reshape(T, H)
    out, logits, counts = _moe_sparse(xf, W_router, W_up, b_up, W_down,
                                      b_down)
    usage = counts[0] * (E / (T * K))
    s = jnp.sum(usage)
    aux_loss = s * s / (E * E)
    return out.reshape(B, S, H), logits, aux_loss
